# baseline (speedup 1.0000x reference)
import jax
import jax.numpy as jnp
from jax.experimental import pallas as pl


def _copy_body(obs_ref, act_ref, obs_out, act_out):
    obs_out[...] = obs_ref[...]
    act_out[...] = act_ref[...]


def kernel(obs_table, act_table):
    n, obs_d = obs_table.shape
    _, act_d = act_table.shape

    fold = 500
    rows = n // fold
    obs2 = obs_table.reshape(rows, fold * obs_d)
    act2 = act_table.reshape(rows, fold * act_d)

    block_rows = 40
    grid = rows // block_rows

    obs_o, act_o = pl.pallas_call(
        _copy_body,
        grid=(grid,),
        in_specs=[
            pl.BlockSpec((block_rows, fold * obs_d), lambda i: (i, 0)),
            pl.BlockSpec((block_rows, fold * act_d), lambda i: (i, 0)),
        ],
        out_specs=[
            pl.BlockSpec((block_rows, fold * obs_d), lambda i: (i, 0)),
            pl.BlockSpec((block_rows, fold * act_d), lambda i: (i, 0)),
        ],
        out_shape=[
            jax.ShapeDtypeStruct(obs2.shape, obs_table.dtype),
            jax.ShapeDtypeStruct(act2.shape, act_table.dtype),
        ],
    )(obs2, act2)

    return (obs_o.reshape(n, obs_d), act_o.reshape(n, act_d))
